# baseline (speedup 1.0000x reference)
import functools

import jax
import jax.numpy as jnp
from jax import lax
from jax.experimental import pallas as pl
from jax.experimental.pallas import tpu as pltpu
from jax.experimental.pallas import tpu_sc as plsc

TIME = 288
FEATURES = 64
PPH = 12
NUM_NODES = 883
B = 64
T = 12
NT = NUM_NODES * T
NW = 32
B_PER_W = B // NW
FG = 8
C_C = 1920
NCH = NT // C_C
TAILC = NT - NCH * C_C


def _sc_body(seq_hbm, td_hbm, tw_hbm, out_hbm, st_v, td_v, tw_v, e_v, buf_v,
             tailb_v, gsem, osem):
    cid = lax.axis_index("c")
    sid = lax.axis_index("s")
    wid = sid * 2 + cid
    lane = lax.broadcasted_iota(jnp.int32, (16,), 0)
    col = jnp.minimum(lane, T - 1)
    pat = [(lane + 16 * j) % T for j in range(3)]
    pat_tail = (lane + (TAILC - 16)) % T

    for k in range(B_PER_W):
        b = wid * B_PER_W + k
        pltpu.sync_copy(seq_hbm.at[b], st_v)
        dayf = plsc.load_gather(st_v, [jnp.full((16,), 2, jnp.int32), col])
        hourf = plsc.load_gather(st_v, [jnp.full((16,), 3, jnp.int32), col])
        minf = plsc.load_gather(st_v, [jnp.full((16,), 4, jnp.int32), col])
        hour = (hourf + 0.5) * 23
        minute = (minf + 0.5) * 59
        hi = ((hour * 60 + minute) / (60.0 / PPH)).astype(jnp.int32)
        hi = jnp.clip(hi, 0, TIME - 1)
        dy = ((dayf + 0.5) * 6).astype(jnp.int32)
        dy = jnp.clip(dy, 0, 6)
        pltpu.async_copy(td_hbm.at[hi], td_v, gsem).wait()
        pltpu.async_copy(tw_hbm.at[dy], tw_v, gsem).wait()
        for r in range(T):
            for cc in range(0, FEATURES, 16):
                e_v[r, pl.ds(cc, 16)] = (td_v[r, pl.ds(cc, 16)] +
                                         tw_v[r, pl.ds(cc, 16)])

        def gbody(g, carry):
            for r in range(FG):
                fv = jnp.full((16,), 0, jnp.int32) + (g * FG + r)
                v = [plsc.load_gather(e_v, [pat[j], fv]) for j in range(3)]
                for i in range(C_C // 48):
                    base = 48 * i
                    buf_v[r, pl.ds(base, 16)] = v[0]
                    buf_v[r, pl.ds(base + 16, 16)] = v[1]
                    buf_v[r, pl.ds(base + 32, 16)] = v[2]
                v3 = plsc.load_gather(e_v, [pat_tail, fv])
                for m in range(TAILC // 16):
                    tailb_v[r, pl.ds(16 * m, 16)] = v[m % 3]
                tailb_v[r, pl.ds(TAILC - 16, 16)] = v3
            cps = [
                pltpu.make_async_copy(
                    buf_v,
                    out_hbm.at[b, pl.ds(g * FG, FG), pl.ds(kk * C_C, C_C)],
                    osem)
                for kk in range(NCH)
            ]
            cps.append(pltpu.make_async_copy(
                tailb_v,
                out_hbm.at[b, pl.ds(g * FG, FG), pl.ds(NCH * C_C, TAILC)],
                osem))
            for cp in cps:
                cp.start()
            for cp in cps:
                cp.wait()
            return carry

        lax.fori_loop(0, FEATURES // FG, gbody, 0)


@functools.partial(jax.jit)
def _sc_call(seq_time, time_day, time_week):
    kfn = pl.kernel(
        _sc_body,
        out_type=jax.ShapeDtypeStruct((B, FEATURES, NT), jnp.float32),
        mesh=plsc.VectorSubcoreMesh(core_axis_name="c", subcore_axis_name="s"),
        scratch_types=[
            pltpu.VMEM((5, T), jnp.float32),
            pltpu.VMEM((16, 128), jnp.float32),
            pltpu.VMEM((16, 128), jnp.float32),
            pltpu.VMEM((16, FEATURES), jnp.float32),
            pltpu.VMEM((FG, C_C), jnp.float32),
            pltpu.VMEM((FG, TAILC), jnp.float32),
            pltpu.SemaphoreType.DMA,
            pltpu.SemaphoreType.DMA,
        ],
        compiler_params=pltpu.CompilerParams(needs_layout_passes=False),
    )
    return kfn(seq_time, time_day, time_week)


def kernel(seq_time, time_day, time_week):
    td_p = jnp.pad(time_day, ((0, 0), (0, 128 - FEATURES)))
    tw_p = jnp.pad(time_week, ((0, 0), (0, 128 - FEATURES)))
    flat = _sc_call(seq_time, td_p, tw_p)
    return flat.reshape(B, FEATURES, NUM_NODES, T)


# speedup vs baseline: 1.2883x; 1.2883x over previous
import functools

import jax
import jax.numpy as jnp
from jax import lax
from jax.experimental import pallas as pl
from jax.experimental.pallas import tpu as pltpu
from jax.experimental.pallas import tpu_sc as plsc

TIME = 288
FEATURES = 64
PPH = 12
NUM_NODES = 883
B = 64
T = 12
NT = NUM_NODES * T
NW = 32
B_PER_W = B // NW
FG = 8
C_C = 1920
NCH = NT // C_C
TAILC = 1024
NTP = NCH * C_C + TAILC


def _sc_body(seq_hbm, td_hbm, tw_hbm, out_hbm, st_v, td_v, tw_v, e_v, buf_a,
             buf_b, gsem, osem):
    cid = lax.axis_index("c")
    sid = lax.axis_index("s")
    wid = sid * 2 + cid
    lane = lax.broadcasted_iota(jnp.int32, (16,), 0)
    col = jnp.minimum(lane, T - 1)
    pat = [(lane + 16 * j) % T for j in range(3)]

    for k in range(B_PER_W):
        b = wid * B_PER_W + k
        pltpu.sync_copy(seq_hbm.at[b], st_v)
        dayf = plsc.load_gather(st_v, [jnp.full((16,), 2, jnp.int32), col])
        hourf = plsc.load_gather(st_v, [jnp.full((16,), 3, jnp.int32), col])
        minf = plsc.load_gather(st_v, [jnp.full((16,), 4, jnp.int32), col])
        hour = (hourf + 0.5) * 23
        minute = (minf + 0.5) * 59
        hi = ((hour * 60 + minute) / (60.0 / PPH)).astype(jnp.int32)
        hi = jnp.clip(hi, 0, TIME - 1)
        dy = ((dayf + 0.5) * 6).astype(jnp.int32)
        dy = jnp.clip(dy, 0, 6)
        pltpu.async_copy(td_hbm.at[hi], td_v, gsem).wait()
        pltpu.async_copy(tw_hbm.at[dy], tw_v, gsem).wait()
        for r in range(T):
            for cc in range(0, FEATURES, 16):
                e_v[r, pl.ds(cc, 16)] = (td_v[r, pl.ds(cc, 16)] +
                                         tw_v[r, pl.ds(cc, 16)])

        def fill(buf, g):
            for r in range(FG):
                fv = jnp.full((16,), 0, jnp.int32) + (g * FG + r)
                v = [plsc.load_gather(e_v, [pat[j], fv]) for j in range(3)]
                for i in range(C_C // 48):
                    base = 48 * i
                    buf[r, pl.ds(base, 16)] = v[0]
                    buf[r, pl.ds(base + 16, 16)] = v[1]
                    buf[r, pl.ds(base + 32, 16)] = v[2]

        def descs(buf, g):
            cps = [
                pltpu.make_async_copy(
                    buf,
                    out_hbm.at[b, pl.ds(g * FG, FG), pl.ds(kk * C_C, C_C)],
                    osem)
                for kk in range(NCH)
            ]
            cps.append(pltpu.make_async_copy(
                buf.at[:, pl.ds(0, TAILC)],
                out_hbm.at[b, pl.ds(NCH * C_C // C_C * 0 + g * FG, FG),
                           pl.ds(NCH * C_C, TAILC)],
                osem))
            return cps

        def gbody(i, carry):
            g0 = i * 2

            @pl.when(i > 0)
            def _drain():
                for cp in descs(buf_a, g0 - 2):
                    cp.wait()
                for cp in descs(buf_b, g0 - 1):
                    cp.wait()

            fill(buf_a, g0)
            for cp in descs(buf_a, g0):
                cp.start()
            fill(buf_b, g0 + 1)
            for cp in descs(buf_b, g0 + 1):
                cp.start()
            return carry

        ngroups = FEATURES // FG
        lax.fori_loop(0, ngroups // 2, gbody, 0)
        for cp in descs(buf_a, ngroups - 2):
            cp.wait()
        for cp in descs(buf_b, ngroups - 1):
            cp.wait()


@functools.partial(jax.jit)
def _sc_call(seq_time, time_day, time_week):
    kfn = pl.kernel(
        _sc_body,
        out_type=jax.ShapeDtypeStruct((B, FEATURES, NTP), jnp.float32),
        mesh=plsc.VectorSubcoreMesh(core_axis_name="c", subcore_axis_name="s"),
        scratch_types=[
            pltpu.VMEM((5, T), jnp.float32),
            pltpu.VMEM((16, 128), jnp.float32),
            pltpu.VMEM((16, 128), jnp.float32),
            pltpu.VMEM((16, FEATURES), jnp.float32),
            pltpu.VMEM((FG, C_C), jnp.float32),
            pltpu.VMEM((FG, C_C), jnp.float32),
            pltpu.SemaphoreType.DMA,
            pltpu.SemaphoreType.DMA,
        ],
        compiler_params=pltpu.CompilerParams(needs_layout_passes=False),
    )
    return kfn(seq_time, time_day, time_week)


def kernel(seq_time, time_day, time_week):
    td_p = jnp.pad(time_day, ((0, 0), (0, 128 - FEATURES)))
    tw_p = jnp.pad(time_week, ((0, 0), (0, 128 - FEATURES)))
    flat = _sc_call(seq_time, td_p, tw_p)
    return flat[:, :, :NT].reshape(B, FEATURES, NUM_NODES, T)


# speedup vs baseline: 1.2889x; 1.0005x over previous
import functools

import jax
import jax.numpy as jnp
from jax import lax
from jax.experimental import pallas as pl
from jax.experimental.pallas import tpu as pltpu
from jax.experimental.pallas import tpu_sc as plsc

TIME = 288
FEATURES = 64
PPH = 12
NUM_NODES = 883
B = 64
T = 12
NT = NUM_NODES * T
NW = 32
B_PER_W = B // NW
FG = 16
C_C = 1920
NCH = NT // C_C
TAILC = 1024
NTP = NCH * C_C + TAILC


def _sc_body(seq_hbm, td_hbm, tw_hbm, out_hbm, st_v, td_v, tw_v, e_v, buf_v,
             gsem, osem):
    cid = lax.axis_index("c")
    sid = lax.axis_index("s")
    wid = sid * 2 + cid
    lane = lax.broadcasted_iota(jnp.int32, (16,), 0)
    col = jnp.minimum(lane, T - 1)
    pat = [(lane + 16 * j) % T for j in range(3)]

    for k in range(B_PER_W):
        b = wid * B_PER_W + k
        pltpu.sync_copy(seq_hbm.at[b], st_v)
        dayf = plsc.load_gather(st_v, [jnp.full((16,), 2, jnp.int32), col])
        hourf = plsc.load_gather(st_v, [jnp.full((16,), 3, jnp.int32), col])
        minf = plsc.load_gather(st_v, [jnp.full((16,), 4, jnp.int32), col])
        hour = (hourf + 0.5) * 23
        minute = (minf + 0.5) * 59
        hi = ((hour * 60 + minute) / (60.0 / PPH)).astype(jnp.int32)
        hi = jnp.clip(hi, 0, TIME - 1)
        dy = ((dayf + 0.5) * 6).astype(jnp.int32)
        dy = jnp.clip(dy, 0, 6)
        pltpu.async_copy(td_hbm.at[hi], td_v, gsem).wait()
        pltpu.async_copy(tw_hbm.at[dy], tw_v, gsem).wait()
        for r in range(T):
            for cc in range(0, FEATURES, 16):
                e_v[r, pl.ds(cc, 16)] = (td_v[r, pl.ds(cc, 16)] +
                                         tw_v[r, pl.ds(cc, 16)])

        def gbody(g, carry):
            for r in range(FG):
                fv = jnp.full((16,), 0, jnp.int32) + (g * FG + r)
                v = [plsc.load_gather(e_v, [pat[j], fv]) for j in range(3)]
                for i in range(C_C // 48):
                    base = 48 * i
                    buf_v[r, pl.ds(base, 16)] = v[0]
                    buf_v[r, pl.ds(base + 16, 16)] = v[1]
                    buf_v[r, pl.ds(base + 32, 16)] = v[2]
            cps = [
                pltpu.make_async_copy(
                    buf_v,
                    out_hbm.at[b, pl.ds(g * FG, FG), pl.ds(kk * C_C, C_C)],
                    osem)
                for kk in range(NCH)
            ]
            cps.append(pltpu.make_async_copy(
                buf_v.at[:, pl.ds(0, TAILC)],
                out_hbm.at[b, pl.ds(g * FG, FG), pl.ds(NCH * C_C, TAILC)],
                osem))
            for cp in cps:
                cp.start()
            for cp in cps:
                cp.wait()
            return carry

        lax.fori_loop(0, FEATURES // FG, gbody, 0)


@functools.partial(jax.jit)
def _sc_call(seq_time, time_day, time_week):
    kfn = pl.kernel(
        _sc_body,
        out_type=jax.ShapeDtypeStruct((B, FEATURES, NTP), jnp.float32),
        mesh=plsc.VectorSubcoreMesh(core_axis_name="c", subcore_axis_name="s"),
        scratch_types=[
            pltpu.VMEM((5, T), jnp.float32),
            pltpu.VMEM((16, 128), jnp.float32),
            pltpu.VMEM((16, 128), jnp.float32),
            pltpu.VMEM((16, FEATURES), jnp.float32),
            pltpu.VMEM((FG, C_C), jnp.float32),
            pltpu.SemaphoreType.DMA,
            pltpu.SemaphoreType.DMA,
        ],
        compiler_params=pltpu.CompilerParams(needs_layout_passes=False),
    )
    return kfn(seq_time, time_day, time_week)


def kernel(seq_time, time_day, time_week):
    td_p = jnp.pad(time_day, ((0, 0), (0, 128 - FEATURES)))
    tw_p = jnp.pad(time_week, ((0, 0), (0, 128 - FEATURES)))
    flat = _sc_call(seq_time, td_p, tw_p)
    return flat[:, :, :NT].reshape(B, FEATURES, NUM_NODES, T)


# speedup vs baseline: 1.3001x; 1.0087x over previous
import functools

import jax
import jax.numpy as jnp
from jax import lax
from jax.experimental import pallas as pl
from jax.experimental.pallas import tpu as pltpu
from jax.experimental.pallas import tpu_sc as plsc

TIME = 288
FEATURES = 64
PPH = 12
NUM_NODES = 883
B = 64
T = 12
NT = NUM_NODES * T
NW = 32
B_PER_W = B // NW
FG = 8
C_C = 1920
NCH = NT // C_C
TAILC = 1024
NTP = NCH * C_C + TAILC


def _sc_body(seq_hbm, td_hbm, tw_hbm, out_hbm, st_v, td_v, tw_v, e_v, buf_v,
             gsem, osem):
    cid = lax.axis_index("c")
    sid = lax.axis_index("s")
    wid = sid * 2 + cid
    lane = lax.broadcasted_iota(jnp.int32, (16,), 0)
    col = jnp.minimum(lane, T - 1)
    pat = [(lane + 16 * j) % T for j in range(3)]

    for k in range(B_PER_W):
        b = wid * B_PER_W + k
        pltpu.sync_copy(seq_hbm.at[b], st_v)
        dayf = plsc.load_gather(st_v, [jnp.full((16,), 2, jnp.int32), col])
        hourf = plsc.load_gather(st_v, [jnp.full((16,), 3, jnp.int32), col])
        minf = plsc.load_gather(st_v, [jnp.full((16,), 4, jnp.int32), col])
        hour = (hourf + 0.5) * 23
        minute = (minf + 0.5) * 59
        hi = ((hour * 60 + minute) / (60.0 / PPH)).astype(jnp.int32)
        hi = jnp.clip(hi, 0, TIME - 1)
        dy = ((dayf + 0.5) * 6).astype(jnp.int32)
        dy = jnp.clip(dy, 0, 6)
        pltpu.async_copy(td_hbm.at[hi], td_v, gsem).wait()
        pltpu.async_copy(tw_hbm.at[dy], tw_v, gsem).wait()
        for r in range(T):
            for cc in range(0, FEATURES, 16):
                e_v[r, pl.ds(cc, 16)] = (td_v[r, pl.ds(cc, 16)] +
                                         tw_v[r, pl.ds(cc, 16)])

        def gbody(g, carry):
            for r in range(FG):
                fv = jnp.full((16,), 0, jnp.int32) + (g * FG + r)
                v = [plsc.load_gather(e_v, [pat[j], fv]) for j in range(3)]
                for i in range(C_C // 48):
                    base = 48 * i
                    buf_v[r, pl.ds(base, 16)] = v[0]
                    buf_v[r, pl.ds(base + 16, 16)] = v[1]
                    buf_v[r, pl.ds(base + 32, 16)] = v[2]
            cps = [
                pltpu.make_async_copy(
                    buf_v,
                    out_hbm.at[b, pl.ds(g * FG, FG), pl.ds(kk * C_C, C_C)],
                    osem)
                for kk in range(NCH)
            ]
            cps.append(pltpu.make_async_copy(
                buf_v.at[:, pl.ds(0, TAILC)],
                out_hbm.at[b, pl.ds(g * FG, FG), pl.ds(NCH * C_C, TAILC)],
                osem))
            for cp in cps:
                cp.start()
            for cp in cps:
                cp.wait()
            return carry

        lax.fori_loop(0, FEATURES // FG, gbody, 0)


@functools.partial(jax.jit)
def _sc_call(seq_time, time_day, time_week):
    kfn = pl.kernel(
        _sc_body,
        out_type=jax.ShapeDtypeStruct((B, FEATURES, NTP), jnp.float32),
        mesh=plsc.VectorSubcoreMesh(core_axis_name="c", subcore_axis_name="s"),
        scratch_types=[
            pltpu.VMEM((5, T), jnp.float32),
            pltpu.VMEM((16, 128), jnp.float32),
            pltpu.VMEM((16, 128), jnp.float32),
            pltpu.VMEM((16, FEATURES), jnp.float32),
            pltpu.VMEM((FG, C_C), jnp.float32),
            pltpu.SemaphoreType.DMA,
            pltpu.SemaphoreType.DMA,
        ],
        compiler_params=pltpu.CompilerParams(needs_layout_passes=False),
    )
    return kfn(seq_time, time_day, time_week)


def kernel(seq_time, time_day, time_week):
    td_p = jnp.pad(time_day, ((0, 0), (0, 128 - FEATURES)))
    tw_p = jnp.pad(time_week, ((0, 0), (0, 128 - FEATURES)))
    flat = _sc_call(seq_time, td_p, tw_p)
    return flat[:, :, :NT].reshape(B, FEATURES, NUM_NODES, T)
